# baseline (speedup 1.0000x reference)
import functools

import jax
import jax.numpy as jnp
from jax import lax
from jax.experimental import pallas as pl
from jax.experimental.pallas import tpu as pltpu

_BN_EPS = 1e-5
_IMG_BLOCK = 16
_VMEM_LIMIT = 100 * 1024 * 1024


def _conv_body(x_ref, w_ref, scale_ref, shift_ref, y_ref, stats_ref,
               *maybe_scratch, H, W, Ci, Co, B, fuse, pad_out):
    Hp2 = H + 2
    M = B * Hp2 - 2
    if fuse:
        a = x_ref[...]
        a = jnp.maximum(a * scale_ref[...] + shift_ref[...], 0.0)
        r = lax.broadcasted_iota(jnp.int32, a.shape, 0) % Hp2
        a = jnp.where((r >= 1) & (r <= H), a, 0.0)
        scratch = maybe_scratch[0]
        scratch[...] = a.astype(w_ref.dtype)
        src = scratch
    else:
        src = x_ref
    a0 = src[pl.ds(0, M), :]
    a1 = src[pl.ds(1, M), :]
    a2 = src[pl.ds(2, M), :]

    valid = (lax.broadcasted_iota(jnp.int32, (M, 2 * Co), 0) % Hp2) < H
    outs, s1s, s2s = [], [], []
    for p in range(W // 2):
        sl = slice(p * 2 * Ci, p * 2 * Ci + 4 * Ci)
        acc = jnp.dot(a0[:, sl], w_ref[0], preferred_element_type=jnp.float32)
        acc = acc + jnp.dot(a1[:, sl], w_ref[1],
                            preferred_element_type=jnp.float32)
        acc = acc + jnp.dot(a2[:, sl], w_ref[2],
                            preferred_element_type=jnp.float32)
        outs.append(acc)
        am = jnp.where(valid, acc, 0.0)
        s1s.append(jnp.sum(am, axis=0, keepdims=True))
        s2s.append(jnp.sum(am * am, axis=0, keepdims=True))

    y = jnp.concatenate(outs, axis=1)
    if pad_out:
        y = jnp.pad(y.astype(y_ref.dtype), ((1, 1), (0, 0)))
        rr = lax.broadcasted_iota(jnp.int32, y.shape, 0) % Hp2
        y = jnp.where((rr >= 1) & (rr <= H), y, jnp.zeros_like(y))
        z = jnp.zeros((B * Hp2, Co), y.dtype)
        y_ref[...] = jnp.concatenate([z, y, z], axis=1)
    else:
        y = jnp.pad(y, ((0, 2), (0, 0)))
        y_ref[...] = y.reshape(B, Hp2, W * Co)[:, :H, :].astype(y_ref.dtype)
    stats_ref[0] = jnp.concatenate(
        [jnp.concatenate(s1s, axis=1), jnp.concatenate(s2s, axis=1)], axis=0)


def _conv_pass(xp, wt, scale_b, shift_b, *, N, H, W, Ci, Co, fuse,
               out_dtype=jnp.float32, pad_out=False):
    Hp2 = H + 2
    Kw = xp.shape[1]
    B = _IMG_BLOCK if N % _IMG_BLOCK == 0 else 1
    G = N // B
    R = B * Hp2
    Wco = W * Co
    if scale_b is None:
        scale_b = jnp.zeros((1, Kw), jnp.float32)
        shift_b = jnp.zeros((1, Kw), jnp.float32)
    if pad_out:
        y_shape, y_block = (N * Hp2, (W + 2) * Co), (R, (W + 2) * Co)
        y_index = lambda n: (n, 0)
    else:
        y_shape, y_block = (N, H, Wco), (B, H, Wco)
        y_index = lambda n: (n, 0, 0)
    body = functools.partial(_conv_body, H=H, W=W, Ci=Ci, Co=Co, B=B,
                             fuse=fuse, pad_out=pad_out)
    return pl.pallas_call(
        body,
        out_shape=(jax.ShapeDtypeStruct(y_shape, out_dtype),
                   jax.ShapeDtypeStruct((G, 2, Wco), jnp.float32)),
        grid=(G,),
        in_specs=[pl.BlockSpec((R, Kw), lambda n: (n, 0)),
                  pl.BlockSpec(wt.shape, lambda n: (0, 0, 0)),
                  pl.BlockSpec((1, Kw), lambda n: (0, 0)),
                  pl.BlockSpec((1, Kw), lambda n: (0, 0))],
        out_specs=(pl.BlockSpec(y_block, y_index),
                   pl.BlockSpec((1, 2, Wco), lambda n: (n, 0, 0))),
        scratch_shapes=(
            [pltpu.VMEM((R, Kw), jnp.bfloat16)] if fuse else []),
        compiler_params=pltpu.CompilerParams(
            dimension_semantics=("parallel",),
            vmem_limit_bytes=_VMEM_LIMIT),
    )(xp, wt, scale_b, shift_b)


def _bn_body(y_ref, scale_ref, shift_ref, o_ref):
    v = jnp.maximum(y_ref[...] * scale_ref[...] + shift_ref[...], 0.0)
    o_ref[...] = v.astype(o_ref.dtype)


def _bn_relu(y2d, scale_row, shift_row):
    rows, Wc = y2d.shape
    br = rows
    for cand in (1024, 512, 256, 128, 64, 32, 16, 8):
        if rows % cand == 0:
            br = cand
            break
    return pl.pallas_call(
        _bn_body,
        out_shape=jax.ShapeDtypeStruct((rows, Wc), y2d.dtype),
        grid=(rows // br,),
        in_specs=[pl.BlockSpec((br, Wc), lambda i: (i, 0)),
                  pl.BlockSpec((1, Wc), lambda i: (0, 0)),
                  pl.BlockSpec((1, Wc), lambda i: (0, 0))],
        out_specs=pl.BlockSpec((br, Wc), lambda i: (i, 0)),
        compiler_params=pltpu.CompilerParams(
            dimension_semantics=("parallel",),
            vmem_limit_bytes=_VMEM_LIMIT),
    )(y2d, scale_row, shift_row)


def _pair_weight(w):
    Co, Ci = w.shape[0], w.shape[1]
    wt = jnp.transpose(w, (2, 1, 3, 0))
    blk = jnp.zeros((3, 4, Ci, 2, Co), w.dtype)
    for v in range(2):
        for dx in range(3):
            blk = blk.at[:, v + dx, :, v, :].set(wt[:, :, dx, :])
    return blk.reshape(3, 4 * Ci, 2 * Co)


def _fold_bn(stats, gamma, beta, count, W, Co):
    t = jnp.sum(stats, axis=0).reshape(2, W, Co).sum(axis=1)
    mean = t[0] / count
    var = jnp.maximum(t[1] / count - mean * mean, 0.0)
    scale = gamma * lax.rsqrt(var + _BN_EPS)
    return scale, beta - mean * scale


def _lane_row(v, W, bordered):
    r = jnp.tile(v, W)
    if bordered:
        z = jnp.zeros_like(v)
        r = jnp.concatenate([z, r, z])
    return r[None, :]


@jax.jit
def _forward(x, w1, w2, g1, be1, g2, be2):
    N, Ci, H, W = x.shape
    C1, C2 = w1.shape[0], w2.shape[0]

    xl = jnp.transpose(x, (0, 2, 3, 1)).reshape(N, H, W * Ci)
    xp = jnp.pad(xl.astype(jnp.bfloat16), ((0, 0), (1, 1), (Ci, Ci)))
    xp = xp.reshape(N * (H + 2), (W + 2) * Ci)

    y1p, st1 = _conv_pass(xp, _pair_weight(w1).astype(jnp.bfloat16), None, None,
                          N=N, H=H, W=W, Ci=Ci, Co=C1, fuse=False,
                          out_dtype=jnp.bfloat16, pad_out=True)
    sc1, sh1 = _fold_bn(st1, g1, be1, N * H * W, W, C1)

    y2, st2 = _conv_pass(y1p, _pair_weight(w2).astype(jnp.bfloat16),
                         _lane_row(sc1, W, True), _lane_row(sh1, W, True),
                         N=N, H=H, W=W, Ci=C1, Co=C2, fuse=True,
                         out_dtype=jnp.bfloat16)
    sc2, sh2 = _fold_bn(st2, g2, be2, N * H * W, W, C2)

    o = _bn_relu(y2.reshape(N * H, W * C2),
                 _lane_row(sc2, W, False), _lane_row(sh2, W, False))
    o = o.reshape(N, H, W, C2).astype(jnp.float32)
    return jnp.transpose(o, (0, 3, 1, 2))


def kernel(x, w1, b1, w2, b2, g1, be1, g2, be2):
    return _forward(x, w1, w2, g1, be1, g2, be2)


# speedup vs baseline: 1.0765x; 1.0765x over previous
import functools

import jax
import jax.numpy as jnp
from jax import lax
from jax.experimental import pallas as pl
from jax.experimental.pallas import tpu as pltpu

_BN_EPS = 1e-5
_IMG_BLOCK = 16
_VMEM_LIMIT = 100 * 1024 * 1024


def _fold_rows(stats, gamma, beta, count, Co, W, bordered, dtype):
    t = jnp.sum(stats, axis=0)
    t2 = t[:, :Co] + t[:, Co:2 * Co]
    inv = 1.0 / count
    mean = t2[0:1] * inv
    var = jnp.maximum(t2[1:2] * inv - mean * mean, 0.0)
    scale = gamma * lax.rsqrt(var + _BN_EPS)
    shift = beta - mean * scale
    srow = jnp.tile(scale, (1, W)).astype(dtype)
    hrow = jnp.tile(shift, (1, W)).astype(dtype)
    if bordered:
        z = jnp.zeros((1, Co), dtype)
        srow = jnp.concatenate([z, srow, z], axis=1)
        hrow = jnp.concatenate([z, hrow, z], axis=1)
    return srow, hrow


def _conv_body(x_ref, w_ref, pstats_ref, g_ref, b_ref, y_ref, stats_ref,
               *, H, W, Ci, Co, B, fuse, count):
    Hp2 = H + 2
    a = x_ref[...].reshape(B * Hp2, (W + 2) * Ci)
    if fuse:
        srow, hrow = _fold_rows(pstats_ref[...], g_ref[...], b_ref[...],
                                count, Ci, W, True, jnp.float32)
        a = jnp.maximum(a * srow + hrow, 0.0)
        r = lax.broadcasted_iota(jnp.int32, a.shape, 0) % Hp2
        a = jnp.where((r >= 1) & (r <= H), a, 0.0)
    a = a.astype(w_ref.dtype)

    M = B * Hp2 - 2
    valid = (lax.broadcasted_iota(jnp.int32, (M, 2 * Co), 0) % Hp2) < H
    outs = []
    s1 = jnp.zeros((1, 2 * Co), jnp.float32)
    s2 = jnp.zeros((1, 2 * Co), jnp.float32)
    for p in range(W // 2):
        ap = a[:, p * 2 * Ci: p * 2 * Ci + 4 * Ci]
        acc = jnp.dot(ap[0:M], w_ref[0], preferred_element_type=jnp.float32)
        acc = acc + jnp.dot(ap[1:M + 1], w_ref[1],
                            preferred_element_type=jnp.float32)
        acc = acc + jnp.dot(ap[2:M + 2], w_ref[2],
                            preferred_element_type=jnp.float32)
        outs.append(acc)
        am = jnp.where(valid, acc, 0.0)
        s1 = s1 + jnp.sum(am, axis=0, keepdims=True)
        s2 = s2 + jnp.sum(am * am, axis=0, keepdims=True)
    stats_ref[0] = jnp.concatenate([s1, s2], axis=0)

    y = jnp.concatenate(outs, axis=1)
    if fuse:
        y = jnp.pad(y, ((0, 2), (0, 0)))
        y_ref[...] = y.reshape(B, Hp2, W * Co)[:, :H, :].astype(y_ref.dtype)
    else:
        y = jnp.pad(y.astype(y_ref.dtype), ((1, 1), (0, 0)))
        rr = lax.broadcasted_iota(jnp.int32, y.shape, 0) % Hp2
        y = jnp.where((rr >= 1) & (rr <= H), y, jnp.zeros_like(y))
        z = jnp.zeros((B * Hp2, Co), y.dtype)
        y = jnp.concatenate([z, y, z], axis=1)
        y_ref[...] = y.reshape(B, Hp2, (W + 2) * Co)


def _conv_pass(xp, wt, pstats, g, b, *, H, W, Ci, Co, fuse, count,
               out_dtype=jnp.float32):
    N, Hp2, Kw = xp.shape
    B = _IMG_BLOCK if N % _IMG_BLOCK == 0 else 1
    G = N // B
    Wco = W * Co
    if pstats is None:
        pstats = jnp.zeros((1, 2, 2 * Ci), jnp.float32)
        g = jnp.zeros((1, Ci), jnp.float32)
        b = jnp.zeros((1, Ci), jnp.float32)
    if fuse:
        y_shape, y_block = (N, H, Wco), (B, H, Wco)
    else:
        y_shape, y_block = (N, Hp2, (W + 2) * Co), (B, Hp2, (W + 2) * Co)
    body = functools.partial(_conv_body, H=H, W=W, Ci=Ci, Co=Co, B=B,
                             fuse=fuse, count=count)
    return pl.pallas_call(
        body,
        out_shape=(jax.ShapeDtypeStruct(y_shape, out_dtype),
                   jax.ShapeDtypeStruct((G, 2, 2 * Co), jnp.float32)),
        grid=(G,),
        in_specs=[pl.BlockSpec((B, Hp2, Kw), lambda n: (n, 0, 0)),
                  pl.BlockSpec(wt.shape, lambda n: (0, 0, 0)),
                  pl.BlockSpec(pstats.shape, lambda n: (0, 0, 0)),
                  pl.BlockSpec((1, Ci), lambda n: (0, 0)),
                  pl.BlockSpec((1, Ci), lambda n: (0, 0))],
        out_specs=(pl.BlockSpec(y_block, lambda n: (n, 0, 0)),
                   pl.BlockSpec((1, 2, 2 * Co), lambda n: (n, 0, 0))),
        compiler_params=pltpu.CompilerParams(
            dimension_semantics=("parallel",),
            vmem_limit_bytes=_VMEM_LIMIT),
    )(xp, wt, pstats, g, b)


def _bn_body(y_ref, pstats_ref, g_ref, b_ref, o_ref, *, W, Co, count):
    srow, hrow = _fold_rows(pstats_ref[...], g_ref[...], b_ref[...],
                            count, Co, W, False, jnp.float32)
    v = jnp.maximum(y_ref[...] * srow + hrow, 0.0)
    o_ref[...] = v.astype(o_ref.dtype)


def _bn_relu(y2d, pstats, g, b, *, W, Co, count):
    rows, Wc = y2d.shape
    br = rows
    for cand in (1024, 512, 256, 128, 64, 32, 16, 8):
        if rows % cand == 0:
            br = cand
            break
    body = functools.partial(_bn_body, W=W, Co=Co, count=count)
    return pl.pallas_call(
        body,
        out_shape=jax.ShapeDtypeStruct((rows, Wc), y2d.dtype),
        grid=(rows // br,),
        in_specs=[pl.BlockSpec((br, Wc), lambda i: (i, 0)),
                  pl.BlockSpec(pstats.shape, lambda i: (0, 0, 0)),
                  pl.BlockSpec((1, Co), lambda i: (0, 0)),
                  pl.BlockSpec((1, Co), lambda i: (0, 0))],
        out_specs=pl.BlockSpec((br, Wc), lambda i: (i, 0)),
        compiler_params=pltpu.CompilerParams(
            dimension_semantics=("parallel",),
            vmem_limit_bytes=_VMEM_LIMIT),
    )(y2d, pstats, g, b)


def _pair_weight(w):
    Co, Ci = w.shape[0], w.shape[1]
    wt = jnp.transpose(w, (2, 1, 3, 0))
    blk = jnp.zeros((3, 4, Ci, 2, Co), w.dtype)
    for v in range(2):
        for dx in range(3):
            blk = blk.at[:, v + dx, :, v, :].set(wt[:, :, dx, :])
    return blk.reshape(3, 4 * Ci, 2 * Co).astype(jnp.bfloat16)


@jax.jit
def _forward(x, w1, w2, g1, be1, g2, be2):
    N, Ci, H, W = x.shape
    C1, C2 = w1.shape[0], w2.shape[0]
    cnt = N * H * W

    xl = jnp.transpose(x, (0, 2, 3, 1)).reshape(N, H, W * Ci)
    xp = jnp.pad(xl.astype(jnp.bfloat16), ((0, 0), (1, 1), (Ci, Ci)))

    y1p, st1 = _conv_pass(xp, _pair_weight(w1), None, None, None,
                          H=H, W=W, Ci=Ci, Co=C1, fuse=False, count=cnt,
                          out_dtype=jnp.bfloat16)
    y2, st2 = _conv_pass(y1p, _pair_weight(w2), st1,
                         g1.reshape(1, C1), be1.reshape(1, C1),
                         H=H, W=W, Ci=C1, Co=C2, fuse=True, count=cnt,
                         out_dtype=jnp.bfloat16)
    o = _bn_relu(y2.reshape(N * H, W * C2), st2,
                 g2.reshape(1, C2), be2.reshape(1, C2),
                 W=W, Co=C2, count=cnt)
    o = o.reshape(N, H, W, C2).astype(jnp.float32)
    return jnp.transpose(o, (0, 3, 1, 2))


def kernel(x, w1, b1, w2, b2, g1, be1, g2, be2):
    return _forward(x, w1, w2, g1, be1, g2, be2)


# speedup vs baseline: 1.0865x; 1.0093x over previous
import functools

import jax
import jax.numpy as jnp
from jax import lax
from jax.experimental import pallas as pl
from jax.experimental.pallas import tpu as pltpu

_BN_EPS = 1e-5
_IMG_BLOCK = 32
_VMEM_LIMIT = 100 * 1024 * 1024


def _fold_rows(stats, gamma, beta, count, Co, W, bordered, dtype):
    t = jnp.sum(stats, axis=0)
    t2 = t[:, :Co] + t[:, Co:2 * Co]
    inv = 1.0 / count
    mean = t2[0:1] * inv
    var = jnp.maximum(t2[1:2] * inv - mean * mean, 0.0)
    scale = gamma * lax.rsqrt(var + _BN_EPS)
    shift = beta - mean * scale
    srow = jnp.tile(scale, (1, W)).astype(dtype)
    hrow = jnp.tile(shift, (1, W)).astype(dtype)
    if bordered:
        z = jnp.zeros((1, Co), dtype)
        srow = jnp.concatenate([z, srow, z], axis=1)
        hrow = jnp.concatenate([z, hrow, z], axis=1)
    return srow, hrow


def _conv_body(x_ref, w_ref, pstats_ref, g_ref, b_ref, y_ref, stats_ref,
               *, H, W, Ci, Co, B, fuse, count):
    Hp2 = H + 2
    a = x_ref[...].reshape(B * Hp2, (W + 2) * Ci)
    if fuse:
        srow, hrow = _fold_rows(pstats_ref[...], g_ref[...], b_ref[...],
                                count, Ci, W, True, jnp.float32)
        a = jnp.maximum(a * srow + hrow, 0.0)
        r = lax.broadcasted_iota(jnp.int32, a.shape, 0) % Hp2
        a = jnp.where((r >= 1) & (r <= H), a, 0.0)
    a = a.astype(w_ref.dtype)

    M = B * Hp2 - 2
    valid = (lax.broadcasted_iota(jnp.int32, (M, 2 * Co), 0) % Hp2) < H
    outs = []
    s1 = jnp.zeros((1, 2 * Co), jnp.float32)
    s2 = jnp.zeros((1, 2 * Co), jnp.float32)
    for p in range(W // 2):
        ap = a[:, p * 2 * Ci: p * 2 * Ci + 4 * Ci]
        acc = jnp.dot(ap[0:M], w_ref[0], preferred_element_type=jnp.float32)
        acc = acc + jnp.dot(ap[1:M + 1], w_ref[1],
                            preferred_element_type=jnp.float32)
        acc = acc + jnp.dot(ap[2:M + 2], w_ref[2],
                            preferred_element_type=jnp.float32)
        outs.append(acc)
        am = jnp.where(valid, acc, 0.0)
        s1 = s1 + jnp.sum(am, axis=0, keepdims=True)
        s2 = s2 + jnp.sum(am * am, axis=0, keepdims=True)
    stats_ref[0] = jnp.concatenate([s1, s2], axis=0)

    y = jnp.concatenate(outs, axis=1)
    if fuse:
        y = jnp.pad(y, ((0, 2), (0, 0)))
        y_ref[...] = y.reshape(B, Hp2, W * Co)[:, :H, :].astype(y_ref.dtype)
    else:
        y = jnp.pad(y.astype(y_ref.dtype), ((1, 1), (0, 0)))
        rr = lax.broadcasted_iota(jnp.int32, y.shape, 0) % Hp2
        y = jnp.where((rr >= 1) & (rr <= H), y, jnp.zeros_like(y))
        z = jnp.zeros((B * Hp2, Co), y.dtype)
        y = jnp.concatenate([z, y, z], axis=1)
        y_ref[...] = y.reshape(B, Hp2, (W + 2) * Co)


def _conv_pass(xp, wt, pstats, g, b, *, H, W, Ci, Co, fuse, count,
               out_dtype=jnp.float32):
    N, Hp2, Kw = xp.shape
    B = _IMG_BLOCK if N % _IMG_BLOCK == 0 else 1
    G = N // B
    Wco = W * Co
    if pstats is None:
        pstats = jnp.zeros((1, 2, 2 * Ci), jnp.float32)
        g = jnp.zeros((1, Ci), jnp.float32)
        b = jnp.zeros((1, Ci), jnp.float32)
    if fuse:
        y_shape, y_block = (N, H, Wco), (B, H, Wco)
    else:
        y_shape, y_block = (N, Hp2, (W + 2) * Co), (B, Hp2, (W + 2) * Co)
    body = functools.partial(_conv_body, H=H, W=W, Ci=Ci, Co=Co, B=B,
                             fuse=fuse, count=count)
    return pl.pallas_call(
        body,
        out_shape=(jax.ShapeDtypeStruct(y_shape, out_dtype),
                   jax.ShapeDtypeStruct((G, 2, 2 * Co), jnp.float32)),
        grid=(G,),
        in_specs=[pl.BlockSpec((B, Hp2, Kw), lambda n: (n, 0, 0)),
                  pl.BlockSpec(wt.shape, lambda n: (0, 0, 0)),
                  pl.BlockSpec(pstats.shape, lambda n: (0, 0, 0)),
                  pl.BlockSpec((1, Ci), lambda n: (0, 0)),
                  pl.BlockSpec((1, Ci), lambda n: (0, 0))],
        out_specs=(pl.BlockSpec(y_block, lambda n: (n, 0, 0)),
                   pl.BlockSpec((1, 2, 2 * Co), lambda n: (n, 0, 0))),
        compiler_params=pltpu.CompilerParams(
            dimension_semantics=("parallel",),
            vmem_limit_bytes=_VMEM_LIMIT),
    )(xp, wt, pstats, g, b)


def _bn_body(y_ref, pstats_ref, g_ref, b_ref, o_ref, *, W, Co, count):
    srow, hrow = _fold_rows(pstats_ref[...], g_ref[...], b_ref[...],
                            count, Co, W, False, jnp.float32)
    v = jnp.maximum(y_ref[...] * srow + hrow, 0.0)
    o_ref[...] = v.astype(o_ref.dtype)


def _bn_relu(y2d, pstats, g, b, *, W, Co, count):
    rows, Wc = y2d.shape
    br = rows
    for cand in (1024, 512, 256, 128, 64, 32, 16, 8):
        if rows % cand == 0:
            br = cand
            break
    body = functools.partial(_bn_body, W=W, Co=Co, count=count)
    return pl.pallas_call(
        body,
        out_shape=jax.ShapeDtypeStruct((rows, Wc), y2d.dtype),
        grid=(rows // br,),
        in_specs=[pl.BlockSpec((br, Wc), lambda i: (i, 0)),
                  pl.BlockSpec(pstats.shape, lambda i: (0, 0, 0)),
                  pl.BlockSpec((1, Co), lambda i: (0, 0)),
                  pl.BlockSpec((1, Co), lambda i: (0, 0))],
        out_specs=pl.BlockSpec((br, Wc), lambda i: (i, 0)),
        compiler_params=pltpu.CompilerParams(
            dimension_semantics=("parallel",),
            vmem_limit_bytes=_VMEM_LIMIT),
    )(y2d, pstats, g, b)


def _pair_weight(w):
    Co, Ci = w.shape[0], w.shape[1]
    wt = jnp.transpose(w, (2, 1, 3, 0))
    blk = jnp.zeros((3, 4, Ci, 2, Co), w.dtype)
    for v in range(2):
        for dx in range(3):
            blk = blk.at[:, v + dx, :, v, :].set(wt[:, :, dx, :])
    return blk.reshape(3, 4 * Ci, 2 * Co).astype(jnp.bfloat16)


@jax.jit
def _forward(x, w1, w2, g1, be1, g2, be2):
    N, Ci, H, W = x.shape
    C1, C2 = w1.shape[0], w2.shape[0]
    cnt = N * H * W

    xl = jnp.transpose(x, (0, 2, 3, 1)).reshape(N, H, W * Ci)
    xp = jnp.pad(xl.astype(jnp.bfloat16), ((0, 0), (1, 1), (Ci, Ci)))

    y1p, st1 = _conv_pass(xp, _pair_weight(w1), None, None, None,
                          H=H, W=W, Ci=Ci, Co=C1, fuse=False, count=cnt,
                          out_dtype=jnp.bfloat16)
    y2, st2 = _conv_pass(y1p, _pair_weight(w2), st1,
                         g1.reshape(1, C1), be1.reshape(1, C1),
                         H=H, W=W, Ci=C1, Co=C2, fuse=True, count=cnt,
                         out_dtype=jnp.bfloat16)
    o = _bn_relu(y2.reshape(N * H, W * C2), st2,
                 g2.reshape(1, C2), be2.reshape(1, C2),
                 W=W, Co=C2, count=cnt)
    o = o.reshape(N, H, W, C2).astype(jnp.float32)
    return jnp.transpose(o, (0, 3, 1, 2))


def kernel(x, w1, b1, w2, b2, g1, be1, g2, be2):
    return _forward(x, w1, w2, g1, be1, g2, be2)
